# baseline (speedup 1.0000x reference)
import functools

import jax
import jax.numpy as jnp
from jax import lax
from jax.experimental import pallas as pl
from jax.experimental.pallas import tpu as pltpu
from jax.experimental.pallas import tpu_sc as plsc

N_NODES = 10000
D_FEAT = 256
NSEG = 64
NCLS = 10
NCLS_PAD = 16

ACC_W = 272
CH = 16
BR = 80
NBLK = N_NODES // BR
NSUBCH = BR // CH
NCORES = 2
NSUB = 16
NW = NCORES * NSUB
BPW = -(-NBLK // NW)
LANES = 16

_mesh = plsc.VectorSubcoreMesh(core_axis_name="c", subcore_axis_name="s")


@functools.partial(
    pl.kernel,
    out_type=(
        jax.ShapeDtypeStruct((NSUB, NSEG, ACC_W), jnp.float32),
        jax.ShapeDtypeStruct((NSUB, NSEG, ACC_W), jnp.float32),
    ),
    mesh=_mesh,
    scratch_types=[
        pltpu.VMEM((2, BR, D_FEAT), jnp.float32),
        pltpu.VMEM((2, BR), jnp.int32),
        pltpu.VMEM((NSEG, ACC_W), jnp.float32),
        pltpu.SemaphoreType.DMA,
        pltpu.SemaphoreType.DMA,
        pltpu.SemaphoreType.DMA,
        pltpu.SemaphoreType.DMA,
    ],
)
def _segsum_sc(x_hbm, batch_hbm, out0_hbm, out1_hbm, xbuf, bbuf, acc,
               sx0, sx1, sb0, sb1):
    cid = lax.axis_index("c")
    sid = lax.axis_index("s")
    w = cid * NSUB + sid
    sx = (sx0, sx1)
    sb = (sb0, sb1)

    def _zrow(s, carry):
        for t in range(ACC_W // LANES):
            acc[s, pl.ds(t * LANES, LANES)] = jnp.zeros((LANES,), jnp.float32)
        return carry
    lax.fori_loop(0, NSEG, _zrow, 0)

    def _start(j, b):
        bk = w + NW * j

        @pl.when(bk < NBLK)
        def _():
            pltpu.async_copy(x_hbm.at[pl.ds(bk * BR, BR)], xbuf.at[b], sx[b])
            pltpu.async_copy(batch_hbm.at[pl.ds(bk * BR, BR)], bbuf.at[b], sb[b])

    def _wait(b):
        pltpu.make_async_copy(x_hbm.at[pl.ds(0, BR)], xbuf.at[b], sx[b]).wait()
        pltpu.make_async_copy(batch_hbm.at[pl.ds(0, BR)], bbuf.at[b], sb[b]).wait()

    _start(0, 0)
    _start(1, 1)

    def _group(g, carry):
        for b in range(2):
            j = g * 2 + b
            bk = w + NW * j

            @pl.when(bk < NBLK)
            def _():
                _wait(b)
                e0 = jnp.clip(
                    1.0 - lax.iota(jnp.int32, 16).astype(jnp.float32), 0.0, 1.0)

                def _sub(s, carry2):
                    r0 = s * CH
                    bvec = bbuf[b, pl.ds(r0, CH)]
                    uniform = bvec[0] == bvec[CH - 1]

                    @pl.when(uniform)
                    def _():
                        seg = bvec[0]
                        sums = []
                        for t in range(D_FEAT // LANES):
                            sl = pl.ds(t * LANES, LANES)
                            p0 = xbuf[b, r0 + 0, sl] + xbuf[b, r0 + 1, sl]
                            p1 = xbuf[b, r0 + 2, sl] + xbuf[b, r0 + 3, sl]
                            p2 = xbuf[b, r0 + 4, sl] + xbuf[b, r0 + 5, sl]
                            p3 = xbuf[b, r0 + 6, sl] + xbuf[b, r0 + 7, sl]
                            p4 = xbuf[b, r0 + 8, sl] + xbuf[b, r0 + 9, sl]
                            p5 = xbuf[b, r0 + 10, sl] + xbuf[b, r0 + 11, sl]
                            p6 = xbuf[b, r0 + 12, sl] + xbuf[b, r0 + 13, sl]
                            p7 = xbuf[b, r0 + 14, sl] + xbuf[b, r0 + 15, sl]
                            sums.append(((p0 + p1) + (p2 + p3))
                                        + ((p4 + p5) + (p6 + p7)))
                        for t in range(D_FEAT // LANES):
                            plsc.addupdate(
                                acc.at[seg, pl.ds(t * LANES, LANES)], sums[t])
                        plsc.addupdate(acc.at[seg, pl.ds(D_FEAT, LANES)],
                                       e0 * float(CH))

                    @pl.when(jnp.logical_not(uniform))
                    def _():
                        for i in range(CH):
                            seg = bvec[i]

                            def _col(t, carry4, seg=seg, i=i):
                                plsc.addupdate(
                                    acc.at[seg, pl.ds(t * LANES, LANES)],
                                    xbuf[b, r0 + i, pl.ds(t * LANES, LANES)],
                                )
                                return carry4
                            lax.fori_loop(0, D_FEAT // LANES, _col, 0)
                            plsc.addupdate(
                                acc.at[seg, pl.ds(D_FEAT, LANES)], e0)
                    return carry2
                for s in range(NSUBCH):
                    _sub(s, 0)
            _start(j + 2, b)
        return carry
    lax.fori_loop(0, BPW // 2, _group, 0)

    @pl.when(cid == 0)
    def _():
        pltpu.sync_copy(acc, out0_hbm.at[sid])

    @pl.when(cid == 1)
    def _():
        pltpu.sync_copy(acc, out1_hbm.at[sid])


def _pool_linear_tc(p0_ref, p1_ref, w_ref, b_ref, o_ref):
    p = jnp.sum(p0_ref[...], axis=0) + jnp.sum(p1_ref[...], axis=0)
    cnt = jnp.clip(p[:, D_FEAT:D_FEAT + 1], 1.0, None)
    pooled = p[:, :D_FEAT] / cnt
    res = lax.dot_general(
        w_ref[...], pooled, (((1,), (1,)), ((), ())),
        preferred_element_type=jnp.float32,
    ) + b_ref[...]
    o_ref[...] = res[:NCLS, :]


@jax.jit
def kernel(x, edge_index, batch, W, b):
    del edge_index
    p0, p1 = _segsum_sc(x, batch)
    w_pad = jnp.zeros((NCLS_PAD, D_FEAT), jnp.float32).at[:NCLS].set(W)
    b_pad = jnp.zeros((NCLS_PAD, 1), jnp.float32).at[:NCLS, 0].set(b)
    out_t = pl.pallas_call(
        _pool_linear_tc,
        out_shape=jax.ShapeDtypeStruct((NCLS, NSEG), jnp.float32),
    )(p0, p1, w_pad, b_pad)
    return out_t.T


# speedup vs baseline: 1.8043x; 1.8043x over previous
import functools

import jax
import jax.numpy as jnp
from jax import lax
from jax.experimental import pallas as pl
from jax.experimental.pallas import tpu as pltpu
from jax.experimental.pallas import tpu_sc as plsc

N_NODES = 10000
D_FEAT = 256
NSEG = 64
NCLS = 10
NCLS_PAD = 16

ACC_W = 272
CH = 16
BR = 80
NBLK = N_NODES // BR
NSUBCH = BR // CH
NCORES = 2
NSUB = 16
NW = NCORES * NSUB
BPW = -(-NBLK // NW)
LANES = 16

_mesh = plsc.VectorSubcoreMesh(core_axis_name="c", subcore_axis_name="s")


@functools.partial(
    pl.kernel,
    out_type=(
        jax.ShapeDtypeStruct((NSUB, NSEG, ACC_W), jnp.float32),
        jax.ShapeDtypeStruct((NSUB, NSEG, ACC_W), jnp.float32),
    ),
    mesh=_mesh,
    scratch_types=[
        pltpu.VMEM((2, BR, D_FEAT), jnp.float32),
        pltpu.VMEM((2, BR), jnp.int32),
        pltpu.VMEM((NSEG, ACC_W), jnp.float32),
        pltpu.SemaphoreType.DMA,
        pltpu.SemaphoreType.DMA,
        pltpu.SemaphoreType.DMA,
        pltpu.SemaphoreType.DMA,
    ],
)
def _segsum_sc(x_hbm, batch_hbm, out0_hbm, out1_hbm, xbuf, bbuf, acc,
               sx0, sx1, sb0, sb1):
    cid = lax.axis_index("c")
    sid = lax.axis_index("s")
    w = cid * NSUB + sid
    sx = (sx0, sx1)
    sb = (sb0, sb1)

    def _zrow(s, carry):
        for t in range(ACC_W // LANES):
            acc[s, pl.ds(t * LANES, LANES)] = jnp.zeros((LANES,), jnp.float32)
        return carry
    lax.fori_loop(0, NSEG, _zrow, 0)

    def _start(j, b):
        bk = w + NW * j

        @pl.when(bk < NBLK)
        def _():
            pltpu.async_copy(x_hbm.at[pl.ds(bk * BR, BR)], xbuf.at[b], sx[b])
            pltpu.async_copy(batch_hbm.at[pl.ds(bk * BR, BR)], bbuf.at[b], sb[b])

    def _wait(b):
        pltpu.make_async_copy(x_hbm.at[pl.ds(0, BR)], xbuf.at[b], sx[b]).wait()
        pltpu.make_async_copy(batch_hbm.at[pl.ds(0, BR)], bbuf.at[b], sb[b]).wait()

    _start(0, 0)
    _start(1, 1)

    def _group(g, carry):
        for b in range(2):
            j = g * 2 + b
            bk = w + NW * j

            @pl.when(bk < NBLK)
            def _():
                _wait(b)
                e0 = jnp.clip(
                    1.0 - lax.iota(jnp.int32, 16).astype(jnp.float32), 0.0, 1.0)

                def _sub(s, carry2):
                    r0 = s * CH
                    bvec = bbuf[b, pl.ds(r0, CH)]
                    uniform = bvec[0] == bvec[CH - 1]

                    @pl.when(uniform)
                    def _():
                        seg = bvec[0]

                        @plsc.parallel_loop(0, D_FEAT // LANES, step=1,
                                            unroll=2)
                        def _t(t):
                            sl = pl.ds(t * LANES, LANES)
                            p0 = xbuf[b, r0 + 0, sl] + xbuf[b, r0 + 1, sl]
                            p1 = xbuf[b, r0 + 2, sl] + xbuf[b, r0 + 3, sl]
                            p2 = xbuf[b, r0 + 4, sl] + xbuf[b, r0 + 5, sl]
                            p3 = xbuf[b, r0 + 6, sl] + xbuf[b, r0 + 7, sl]
                            p4 = xbuf[b, r0 + 8, sl] + xbuf[b, r0 + 9, sl]
                            p5 = xbuf[b, r0 + 10, sl] + xbuf[b, r0 + 11, sl]
                            p6 = xbuf[b, r0 + 12, sl] + xbuf[b, r0 + 13, sl]
                            p7 = xbuf[b, r0 + 14, sl] + xbuf[b, r0 + 15, sl]
                            s16 = (((p0 + p1) + (p2 + p3))
                                   + ((p4 + p5) + (p6 + p7)))
                            plsc.addupdate(acc.at[seg, sl], s16)
                        plsc.addupdate(acc.at[seg, pl.ds(D_FEAT, LANES)],
                                       e0 * float(CH))

                    @pl.when(jnp.logical_not(uniform))
                    def _():
                        for i in range(CH):
                            seg = bvec[i]

                            def _col(t, carry4, seg=seg, i=i):
                                plsc.addupdate(
                                    acc.at[seg, pl.ds(t * LANES, LANES)],
                                    xbuf[b, r0 + i, pl.ds(t * LANES, LANES)],
                                )
                                return carry4
                            lax.fori_loop(0, D_FEAT // LANES, _col, 0)
                            plsc.addupdate(
                                acc.at[seg, pl.ds(D_FEAT, LANES)], e0)
                    return carry2
                lax.fori_loop(0, NSUBCH, _sub, 0)
            _start(j + 2, b)
        return carry
    lax.fori_loop(0, BPW // 2, _group, 0)

    @pl.when(cid == 0)
    def _():
        pltpu.sync_copy(acc, out0_hbm.at[sid])

    @pl.when(cid == 1)
    def _():
        pltpu.sync_copy(acc, out1_hbm.at[sid])


def _pool_linear_tc(p0_ref, p1_ref, w_ref, b_ref, o_ref):
    p = jnp.sum(p0_ref[...], axis=0) + jnp.sum(p1_ref[...], axis=0)
    cnt = jnp.clip(p[:, D_FEAT:D_FEAT + 1], 1.0, None)
    pooled = p[:, :D_FEAT] / cnt
    res = lax.dot_general(
        w_ref[...], pooled, (((1,), (1,)), ((), ())),
        preferred_element_type=jnp.float32,
    ) + b_ref[...]
    o_ref[...] = res[:NCLS, :]


@jax.jit
def kernel(x, edge_index, batch, W, b):
    del edge_index
    p0, p1 = _segsum_sc(x, batch)
    w_pad = jnp.zeros((NCLS_PAD, D_FEAT), jnp.float32).at[:NCLS].set(W)
    b_pad = jnp.zeros((NCLS_PAD, 1), jnp.float32).at[:NCLS, 0].set(b)
    out_t = pl.pallas_call(
        _pool_linear_tc,
        out_shape=jax.ShapeDtypeStruct((NCLS, NSEG), jnp.float32),
    )(p0, p1, w_pad, b_pad)
    return out_t.T


# speedup vs baseline: 1.8400x; 1.0198x over previous
import functools

import jax
import jax.numpy as jnp
from jax import lax
from jax.experimental import pallas as pl
from jax.experimental.pallas import tpu as pltpu
from jax.experimental.pallas import tpu_sc as plsc

N_NODES = 10000
D_FEAT = 256
NSEG = 64
NCLS = 10
NCLS_PAD = 16

ACC_W = 272
CH = 16
BR = 80
NBLK = N_NODES // BR
NSUBCH = BR // CH
NCORES = 2
NSUB = 16
NW = NCORES * NSUB
BPW = -(-NBLK // NW)
LANES = 16

_mesh = plsc.VectorSubcoreMesh(core_axis_name="c", subcore_axis_name="s")


@functools.partial(
    pl.kernel,
    out_type=(
        jax.ShapeDtypeStruct((NSUB, NSEG, ACC_W), jnp.float32),
        jax.ShapeDtypeStruct((NSUB, NSEG, ACC_W), jnp.float32),
    ),
    mesh=_mesh,
    scratch_types=[
        pltpu.VMEM((2, BR, D_FEAT), jnp.float32),
        pltpu.VMEM((2, BR), jnp.int32),
        pltpu.VMEM((NSEG, ACC_W), jnp.float32),
        pltpu.SemaphoreType.DMA,
        pltpu.SemaphoreType.DMA,
        pltpu.SemaphoreType.DMA,
        pltpu.SemaphoreType.DMA,
    ],
)
def _segsum_sc(x_hbm, batch_hbm, out0_hbm, out1_hbm, xbuf, bbuf, acc,
               sx0, sx1, sb0, sb1):
    cid = lax.axis_index("c")
    sid = lax.axis_index("s")
    w = cid * NSUB + sid
    sx = (sx0, sx1)
    sb = (sb0, sb1)

    def _zrow(s, carry):
        for t in range(ACC_W // LANES):
            acc[s, pl.ds(t * LANES, LANES)] = jnp.zeros((LANES,), jnp.float32)
        return carry
    lax.fori_loop(0, NSEG, _zrow, 0)

    def _start(j, b):
        bk = w + NW * j

        @pl.when(bk < NBLK)
        def _():
            pltpu.async_copy(x_hbm.at[pl.ds(bk * BR, BR)], xbuf.at[b], sx[b])
            pltpu.async_copy(batch_hbm.at[pl.ds(bk * BR, BR)], bbuf.at[b], sb[b])

    def _wait(b):
        pltpu.make_async_copy(x_hbm.at[pl.ds(0, BR)], xbuf.at[b], sx[b]).wait()
        pltpu.make_async_copy(batch_hbm.at[pl.ds(0, BR)], bbuf.at[b], sb[b]).wait()

    _start(0, 0)
    _start(1, 1)

    def _group(g, carry):
        for b in range(2):
            j = g * 2 + b
            bk = w + NW * j

            @pl.when(bk < NBLK)
            def _():
                _wait(b)
                e0 = jnp.clip(
                    1.0 - lax.iota(jnp.int32, 16).astype(jnp.float32), 0.0, 1.0)

                def _sub(s, carry2):
                    r0 = s * CH
                    bvec = bbuf[b, pl.ds(r0, CH)]
                    uniform = bvec[0] == bvec[CH - 1]

                    @pl.when(uniform)
                    def _():
                        seg = bvec[0]

                        @plsc.parallel_loop(0, D_FEAT // LANES, step=1,
                                            unroll=2)
                        def _t(t):
                            sl = pl.ds(t * LANES, LANES)
                            p0 = xbuf[b, r0 + 0, sl] + xbuf[b, r0 + 1, sl]
                            p1 = xbuf[b, r0 + 2, sl] + xbuf[b, r0 + 3, sl]
                            p2 = xbuf[b, r0 + 4, sl] + xbuf[b, r0 + 5, sl]
                            p3 = xbuf[b, r0 + 6, sl] + xbuf[b, r0 + 7, sl]
                            p4 = xbuf[b, r0 + 8, sl] + xbuf[b, r0 + 9, sl]
                            p5 = xbuf[b, r0 + 10, sl] + xbuf[b, r0 + 11, sl]
                            p6 = xbuf[b, r0 + 12, sl] + xbuf[b, r0 + 13, sl]
                            p7 = xbuf[b, r0 + 14, sl] + xbuf[b, r0 + 15, sl]
                            s16 = (((p0 + p1) + (p2 + p3))
                                   + ((p4 + p5) + (p6 + p7)))
                            plsc.addupdate(acc.at[seg, sl], s16)
                        plsc.addupdate(acc.at[seg, pl.ds(D_FEAT, LANES)],
                                       e0 * float(CH))

                    @pl.when(jnp.logical_not(uniform))
                    def _():
                        for i in range(CH):
                            seg = bvec[i]

                            @plsc.parallel_loop(0, D_FEAT // LANES, step=1,
                                                unroll=2)
                            def _col(t, seg=seg, i=i):
                                plsc.addupdate(
                                    acc.at[seg, pl.ds(t * LANES, LANES)],
                                    xbuf[b, r0 + i, pl.ds(t * LANES, LANES)],
                                )
                            plsc.addupdate(
                                acc.at[seg, pl.ds(D_FEAT, LANES)], e0)
                    return carry2
                lax.fori_loop(0, NSUBCH, _sub, 0)
            _start(j + 2, b)
        return carry
    lax.fori_loop(0, BPW // 2, _group, 0)

    @pl.when(cid == 0)
    def _():
        pltpu.sync_copy(acc, out0_hbm.at[sid])

    @pl.when(cid == 1)
    def _():
        pltpu.sync_copy(acc, out1_hbm.at[sid])


def _pool_linear_tc(p0_ref, p1_ref, w_ref, b_ref, o_ref):
    p = jnp.sum(p0_ref[...], axis=0) + jnp.sum(p1_ref[...], axis=0)
    cnt = jnp.clip(p[:, D_FEAT:D_FEAT + 1], 1.0, None)
    pooled = p[:, :D_FEAT] / cnt
    res = lax.dot_general(
        w_ref[...], pooled, (((1,), (1,)), ((), ())),
        preferred_element_type=jnp.float32,
    ) + b_ref[...]
    o_ref[...] = res[:NCLS, :]


@jax.jit
def kernel(x, edge_index, batch, W, b):
    del edge_index
    p0, p1 = _segsum_sc(x, batch)
    w_pad = jnp.zeros((NCLS_PAD, D_FEAT), jnp.float32).at[:NCLS].set(W)
    b_pad = jnp.zeros((NCLS_PAD, 1), jnp.float32).at[:NCLS, 0].set(b)
    out_t = pl.pallas_call(
        _pool_linear_tc,
        out_shape=jax.ShapeDtypeStruct((NCLS, NSEG), jnp.float32),
    )(p0, p1, w_pad, b_pad)
    return out_t.T
